# Initial kernel scaffold; baseline (speedup 1.0000x reference)
import jax
import jax.numpy as jnp
from jax.experimental import pallas as pl


def kernel(X, W, b):
    raise NotImplementedError("write your pallas kernel here")


# speedup vs baseline: 1.1255x; 1.1255x over previous
import jax
import jax.numpy as jnp
from jax.experimental import pallas as pl
from jax.experimental.pallas import tpu as pltpu

_K = 1024
_TILE = 4096


def _score_body(w_ref, b_ref, x_ref, k_ref):
    x = x_ref[0]
    s = jax.lax.dot_general(w_ref[...], x, (((1,), (0,)), ((), ())),
                            preferred_element_type=jnp.float32)
    s = s + b_ref[0, 0]
    bits = jax.lax.bitcast_convert_type(s, jnp.int32)
    key = jnp.where(bits < 0, bits ^ jnp.int32(0x7FFFFFFF), bits)
    k_ref[...] = key[:, None, None, :].reshape(1, 1, 1, _TILE)


def _select_body(k_ref, thr_ref, cut_ref):
    key = k_ref[...]

    def tbody(i, t):
        cand = t + jax.lax.shift_left(jnp.int32(1), jnp.int32(30) - i)
        cnt = jnp.sum((key >= cand).astype(jnp.int32))
        return jnp.where(cnt >= _K, cand, t)

    n_pos = jnp.sum((key >= 0).astype(jnp.int32))
    t0 = jnp.where(n_pos >= _K, jnp.int32(0), jnp.int32(-2147483648))
    thr = jax.lax.fori_loop(0, 31, tbody, t0)

    n_gt = jnp.sum((key > thr).astype(jnp.int32))
    r = _K - n_gt
    eq = key == thr
    iota = (jax.lax.broadcasted_iota(jnp.int32, key.shape, 1) * _TILE
            + jax.lax.broadcasted_iota(jnp.int32, key.shape, 3))

    def cbody(i, m):
        cand = m + jax.lax.shift_left(jnp.int32(1), jnp.int32(17) - i)
        g = jnp.sum((eq & (iota < cand)).astype(jnp.int32))
        return jnp.where(g < r, cand, m)

    m = jax.lax.fori_loop(0, 18, cbody, jnp.int32(0))
    bi = pl.program_id(0)
    thr_ref[bi, 0] = thr
    cut_ref[bi, 0] = jnp.where(r > 0, m + 1, jnp.int32(0))


def _reduce_body(w_ref, b_ref, thr_ref, cut_ref, x_ref, k_ref,
                 feat_ref, o2_ref):
    bi = pl.program_id(0)
    t = pl.program_id(1)
    nt = pl.num_programs(1)

    @pl.when(t == 0)
    def _():
        feat_ref[...] = jnp.zeros_like(feat_ref)

    key = k_ref[0, 0]
    thr = thr_ref[bi, 0]
    cut = cut_ref[bi, 0]
    iota = jax.lax.broadcasted_iota(jnp.int32, key.shape, 1) + t * _TILE
    sel = (key > thr) | ((key == thr) & (iota < cut))
    wsel = sel.astype(jnp.float32)
    x = x_ref[0]
    part = jax.lax.dot_general(
        wsel, x, (((1,), (1,)), ((), ())),
        precision=jax.lax.Precision.HIGHEST,
        preferred_element_type=jnp.float32)
    feat_ref[...] += part[None]

    @pl.when(t == nt - 1)
    def _():
        f = feat_ref[...] * (1.0 / _K)
        feat_ref[...] = f
        o2_ref[bi, 0] = jnp.sum(f[0] * w_ref[...]) + b_ref[0, 0]


def kernel(X, W, b):
    B, C, H, Wd = X.shape
    HW = H * Wd
    nt = HW // _TILE
    X3 = X.reshape(B, C, HW)
    b2 = b.reshape(1, 1)

    keys = pl.pallas_call(
        _score_body,
        grid=(B, nt),
        in_specs=[
            pl.BlockSpec((1, C), lambda bi, ti: (0, 0)),
            pl.BlockSpec(memory_space=pltpu.SMEM),
            pl.BlockSpec((1, C, _TILE), lambda bi, ti: (bi, 0, ti)),
        ],
        out_specs=pl.BlockSpec((1, 1, 1, _TILE),
                               lambda bi, ti: (bi, ti, 0, 0)),
        out_shape=jax.ShapeDtypeStruct((B, nt, 1, _TILE), jnp.int32),
    )(W, b2, X3)

    thr, cut = pl.pallas_call(
        _select_body,
        grid=(B,),
        in_specs=[pl.BlockSpec((1, nt, 1, _TILE), lambda bi: (bi, 0, 0, 0))],
        out_specs=[
            pl.BlockSpec((B, 1), lambda bi: (0, 0),
                         memory_space=pltpu.SMEM),
            pl.BlockSpec((B, 1), lambda bi: (0, 0),
                         memory_space=pltpu.SMEM),
        ],
        out_shape=[
            jax.ShapeDtypeStruct((B, 1), jnp.int32),
            jax.ShapeDtypeStruct((B, 1), jnp.int32),
        ],
    )(keys)

    feat, out2 = pl.pallas_call(
        _reduce_body,
        grid=(B, nt),
        in_specs=[
            pl.BlockSpec((1, C), lambda bi, ti: (0, 0)),
            pl.BlockSpec(memory_space=pltpu.SMEM),
            pl.BlockSpec((B, 1), lambda bi, ti: (0, 0),
                         memory_space=pltpu.SMEM),
            pl.BlockSpec((B, 1), lambda bi, ti: (0, 0),
                         memory_space=pltpu.SMEM),
            pl.BlockSpec((1, C, _TILE), lambda bi, ti: (bi, 0, ti)),
            pl.BlockSpec((1, 1, 1, _TILE), lambda bi, ti: (bi, ti, 0, 0)),
        ],
        out_specs=[
            pl.BlockSpec((1, 1, C), lambda bi, ti: (bi, 0, 0)),
            pl.BlockSpec((B, 1), lambda bi, ti: (0, 0),
                         memory_space=pltpu.SMEM),
        ],
        out_shape=[
            jax.ShapeDtypeStruct((B, 1, C), jnp.float32),
            jax.ShapeDtypeStruct((B, 1), jnp.float32),
        ],
    )(W, b2, thr, cut, X3, keys)

    return (feat.reshape(B, C), out2)


# speedup vs baseline: 1.2276x; 1.0907x over previous
import jax
import jax.numpy as jnp
from jax.experimental import pallas as pl
from jax.experimental.pallas import tpu as pltpu

_K = 1024
_TILE = 4096


def _score_body(w_ref, b_ref, x_ref, k_ref):
    x = x_ref[0]
    s = jax.lax.dot_general(w_ref[...], x, (((1,), (0,)), ((), ())),
                            preferred_element_type=jnp.float32)
    s = s + b_ref[0, 0]
    bits = jax.lax.bitcast_convert_type(s, jnp.int32)
    key = jnp.where(bits < 0, bits ^ jnp.int32(0x7FFFFFFF), bits)
    k_ref[...] = key[:, None, None, :].reshape(1, 1, 1, _TILE)


def _select_body(k_ref, thr_ref, cut_ref):
    key = k_ref[...]

    def tbody(i, t):
        cand = t + jax.lax.shift_left(jnp.int32(1), jnp.int32(30) - i)
        cnt = jnp.sum((key >= cand).astype(jnp.int32))
        return jnp.where(cnt >= _K, cand, t)

    n_pos = jnp.sum((key >= 0).astype(jnp.int32))
    t0 = jnp.where(n_pos >= _K, jnp.int32(0), jnp.int32(-2147483648))
    thr = jax.lax.fori_loop(0, 31, tbody, t0)

    n_gt = jnp.sum((key > thr).astype(jnp.int32))
    r = _K - n_gt
    eq = key == thr
    iota = (jax.lax.broadcasted_iota(jnp.int32, key.shape, 1) * _TILE
            + jax.lax.broadcasted_iota(jnp.int32, key.shape, 3))

    def cbody(i, m):
        cand = m + jax.lax.shift_left(jnp.int32(1), jnp.int32(17) - i)
        g = jnp.sum((eq & (iota < cand)).astype(jnp.int32))
        return jnp.where(g < r, cand, m)

    m = jax.lax.fori_loop(0, 18, cbody, jnp.int32(0))
    bi = pl.program_id(0)
    thr_ref[bi, 0] = thr
    cut_ref[bi, 0] = jnp.where(r > 0, m + 1, jnp.int32(0))


def _reduce_body(w_ref, b_ref, thr_ref, cut_ref, x_ref, k_ref,
                 feat_ref, o2_ref):
    bi = pl.program_id(0)
    t = pl.program_id(1)
    nt = pl.num_programs(1)

    @pl.when(t == 0)
    def _():
        feat_ref[...] = jnp.zeros_like(feat_ref)

    key = k_ref[0, 0]
    thr = thr_ref[bi, 0]
    cut = cut_ref[bi, 0]
    iota = jax.lax.broadcasted_iota(jnp.int32, key.shape, 1) + t * _TILE
    sel = (key > thr) | ((key == thr) & (iota < cut))
    wsel = sel.astype(jnp.float32)
    x = x_ref[0]
    part = jax.lax.dot_general(
        wsel, x, (((1,), (1,)), ((), ())),
        preferred_element_type=jnp.float32)
    feat_ref[...] += part[None]

    @pl.when(t == nt - 1)
    def _():
        f = feat_ref[...] * (1.0 / _K)
        feat_ref[...] = f
        o2_ref[bi, 0] = jnp.sum(f[0] * w_ref[...]) + b_ref[0, 0]


def kernel(X, W, b):
    B, C, H, Wd = X.shape
    HW = H * Wd
    nt = HW // _TILE
    X3 = X.reshape(B, C, HW)
    b2 = b.reshape(1, 1)

    keys = pl.pallas_call(
        _score_body,
        grid=(B, nt),
        in_specs=[
            pl.BlockSpec((1, C), lambda bi, ti: (0, 0)),
            pl.BlockSpec(memory_space=pltpu.SMEM),
            pl.BlockSpec((1, C, _TILE), lambda bi, ti: (bi, 0, ti)),
        ],
        out_specs=pl.BlockSpec((1, 1, 1, _TILE),
                               lambda bi, ti: (bi, ti, 0, 0)),
        out_shape=jax.ShapeDtypeStruct((B, nt, 1, _TILE), jnp.int32),
    )(W, b2, X3)

    thr, cut = pl.pallas_call(
        _select_body,
        grid=(B,),
        in_specs=[pl.BlockSpec((1, nt, 1, _TILE), lambda bi: (bi, 0, 0, 0))],
        out_specs=[
            pl.BlockSpec((B, 1), lambda bi: (0, 0),
                         memory_space=pltpu.SMEM),
            pl.BlockSpec((B, 1), lambda bi: (0, 0),
                         memory_space=pltpu.SMEM),
        ],
        out_shape=[
            jax.ShapeDtypeStruct((B, 1), jnp.int32),
            jax.ShapeDtypeStruct((B, 1), jnp.int32),
        ],
    )(keys)

    feat, out2 = pl.pallas_call(
        _reduce_body,
        grid=(B, nt),
        in_specs=[
            pl.BlockSpec((1, C), lambda bi, ti: (0, 0)),
            pl.BlockSpec(memory_space=pltpu.SMEM),
            pl.BlockSpec((B, 1), lambda bi, ti: (0, 0),
                         memory_space=pltpu.SMEM),
            pl.BlockSpec((B, 1), lambda bi, ti: (0, 0),
                         memory_space=pltpu.SMEM),
            pl.BlockSpec((1, C, _TILE), lambda bi, ti: (bi, 0, ti)),
            pl.BlockSpec((1, 1, 1, _TILE), lambda bi, ti: (bi, ti, 0, 0)),
        ],
        out_specs=[
            pl.BlockSpec((1, 1, C), lambda bi, ti: (bi, 0, 0)),
            pl.BlockSpec((B, 1), lambda bi, ti: (0, 0),
                         memory_space=pltpu.SMEM),
        ],
        out_shape=[
            jax.ShapeDtypeStruct((B, 1, C), jnp.float32),
            jax.ShapeDtypeStruct((B, 1), jnp.float32),
        ],
    )(W, b2, thr, cut, X3, keys)

    return (feat.reshape(B, C), out2)


# speedup vs baseline: 1.5543x; 1.2661x over previous
import jax
import jax.numpy as jnp
from jax.experimental import pallas as pl
from jax.experimental.pallas import tpu as pltpu

_K = 1024
_TILE = 4096


def _score_body(w_ref, b_ref, x_ref, k_ref):
    x = x_ref[0]
    s = jax.lax.dot_general(w_ref[...], x, (((1,), (0,)), ((), ())),
                            preferred_element_type=jnp.float32)
    s = s + b_ref[0, 0]
    bits = jax.lax.bitcast_convert_type(s, jnp.int32)
    key = jnp.where(bits < 0, bits ^ jnp.int32(0x7FFFFFFF), bits)
    k_ref[...] = key[:, None, None, :].reshape(1, 1, 1, _TILE)


def _select_body(k_ref, thr_ref, cut_ref):
    key = k_ref[...]
    B = key.shape[0]

    def count_ge(c):
        return jnp.sum((key >= c).astype(jnp.int32), axis=1, keepdims=True)

    def tbody(i, t):
        cand = t + jax.lax.shift_left(jnp.int32(1), jnp.int32(30) - i)
        return jnp.where(count_ge(cand) >= _K, cand, t)

    n_pos = count_ge(jnp.zeros((B, 1), jnp.int32))
    t0 = jnp.where(n_pos >= _K, jnp.int32(0), jnp.int32(-2147483648))
    thr = jax.lax.fori_loop(0, 31, tbody, t0)

    n_gt = jnp.sum((key > thr).astype(jnp.int32), axis=1, keepdims=True)
    r = _K - n_gt
    eq = key == thr
    iota = jax.lax.broadcasted_iota(jnp.int32, key.shape, 1)

    def cbody(i, m):
        cand = m + jax.lax.shift_left(jnp.int32(1), jnp.int32(17) - i)
        g = jnp.sum((eq & (iota < cand)).astype(jnp.int32),
                    axis=1, keepdims=True)
        return jnp.where(g < r, cand, m)

    m = jax.lax.fori_loop(0, 18, cbody, jnp.zeros((B, 1), jnp.int32))
    cut = jnp.where(r > 0, m + 1, jnp.int32(0))
    thr_ref[...] = jnp.broadcast_to(thr, thr_ref.shape)
    cut_ref[...] = jnp.broadcast_to(cut, cut_ref.shape)


def _reduce_body(w_ref, b_ref, thr_ref, cut_ref, x_ref, k_ref,
                 feat_ref, o2_ref):
    bi = pl.program_id(0)
    t = pl.program_id(1)
    nt = pl.num_programs(1)

    @pl.when(t == 0)
    def _():
        feat_ref[...] = jnp.zeros_like(feat_ref)

    key = k_ref[0, 0]
    thr = thr_ref[bi, 0]
    cut = cut_ref[bi, 0]
    iota = jax.lax.broadcasted_iota(jnp.int32, key.shape, 1) + t * _TILE
    sel = (key > thr) | ((key == thr) & (iota < cut))
    wsel = sel.astype(jnp.float32)
    x = x_ref[0]
    part = jax.lax.dot_general(
        wsel, x, (((1,), (1,)), ((), ())),
        preferred_element_type=jnp.float32)
    feat_ref[...] += part[None]

    @pl.when(t == nt - 1)
    def _():
        f = feat_ref[...] * (1.0 / _K)
        feat_ref[...] = f
        o2_ref[bi, 0] = jnp.sum(f[0] * w_ref[...]) + b_ref[0, 0]


def kernel(X, W, b):
    B, C, H, Wd = X.shape
    HW = H * Wd
    nt = HW // _TILE
    X3 = X.reshape(B, C, HW)
    b2 = b.reshape(1, 1)

    keys = pl.pallas_call(
        _score_body,
        grid=(B, nt),
        in_specs=[
            pl.BlockSpec((1, C), lambda bi, ti: (0, 0)),
            pl.BlockSpec(memory_space=pltpu.SMEM),
            pl.BlockSpec((1, C, _TILE), lambda bi, ti: (bi, 0, ti)),
        ],
        out_specs=pl.BlockSpec((1, 1, 1, _TILE),
                               lambda bi, ti: (bi, ti, 0, 0)),
        out_shape=jax.ShapeDtypeStruct((B, nt, 1, _TILE), jnp.int32),
    )(W, b2, X3)

    keys2 = keys.reshape(B, HW)

    thr, cut = pl.pallas_call(
        _select_body,
        grid=(1,),
        in_specs=[pl.BlockSpec((B, HW), lambda i: (0, 0))],
        out_specs=[
            pl.BlockSpec((B, 128), lambda i: (0, 0)),
            pl.BlockSpec((B, 128), lambda i: (0, 0)),
        ],
        out_shape=[
            jax.ShapeDtypeStruct((B, 128), jnp.int32),
            jax.ShapeDtypeStruct((B, 128), jnp.int32),
        ],
    )(keys2)

    feat, out2 = pl.pallas_call(
        _reduce_body,
        grid=(B, nt),
        in_specs=[
            pl.BlockSpec((1, C), lambda bi, ti: (0, 0)),
            pl.BlockSpec(memory_space=pltpu.SMEM),
            pl.BlockSpec(memory_space=pltpu.SMEM),
            pl.BlockSpec(memory_space=pltpu.SMEM),
            pl.BlockSpec((1, C, _TILE), lambda bi, ti: (bi, 0, ti)),
            pl.BlockSpec((1, 1, 1, _TILE), lambda bi, ti: (bi, ti, 0, 0)),
        ],
        out_specs=[
            pl.BlockSpec((1, 1, C), lambda bi, ti: (bi, 0, 0)),
            pl.BlockSpec((B, 1), lambda bi, ti: (0, 0),
                         memory_space=pltpu.SMEM),
        ],
        out_shape=[
            jax.ShapeDtypeStruct((B, 1, C), jnp.float32),
            jax.ShapeDtypeStruct((B, 1), jnp.float32),
        ],
    )(W, b2, thr, cut, X3, keys)

    return (feat.reshape(B, C), out2)


# speedup vs baseline: 3.5453x; 2.2810x over previous
import jax
import jax.numpy as jnp
from jax.experimental import pallas as pl
from jax.experimental.pallas import tpu as pltpu

_K = 1024
_TH = 16


def _score_body(w_ref, b_ref, x_ref, k_ref):
    x = x_ref[0]
    xb = x.astype(jnp.bfloat16).astype(jnp.float32)
    wb = w_ref[...].astype(jnp.bfloat16).astype(jnp.float32)
    s = jnp.sum(xb * wb[0][:, None, None], axis=0)
    s = s + b_ref[0, 0]
    bits = jax.lax.bitcast_convert_type(s, jnp.int32)
    key = jnp.where(bits < 0, bits ^ jnp.int32(0x7FFFFFFF), bits)
    k_ref[...] = key[None]


def _select_body(k_ref, thr_ref, cut_ref):
    key = k_ref[...]
    B = key.shape[0]

    def count_ge(c):
        return jnp.sum((key >= c).astype(jnp.int32), axis=1, keepdims=True)

    def tbody(i, t):
        cand = t + jax.lax.shift_left(jnp.int32(1), jnp.int32(30) - i)
        return jnp.where(count_ge(cand) >= _K, cand, t)

    n_pos = count_ge(jnp.zeros((B, 1), jnp.int32))
    t0 = jnp.where(n_pos >= _K, jnp.int32(0), jnp.int32(-2147483648))
    thr = jax.lax.fori_loop(0, 31, tbody, t0)

    n_gt = jnp.sum((key > thr).astype(jnp.int32), axis=1, keepdims=True)
    r = _K - n_gt
    eq = key == thr
    iota = jax.lax.broadcasted_iota(jnp.int32, key.shape, 1)

    def cbody(i, m):
        cand = m + jax.lax.shift_left(jnp.int32(1), jnp.int32(17) - i)
        g = jnp.sum((eq & (iota < cand)).astype(jnp.int32),
                    axis=1, keepdims=True)
        return jnp.where(g < r, cand, m)

    m = jax.lax.fori_loop(0, 18, cbody, jnp.zeros((B, 1), jnp.int32))
    cut = jnp.where(r > 0, m + 1, jnp.int32(0))
    thr_ref[...] = jnp.broadcast_to(thr, thr_ref.shape)
    cut_ref[...] = jnp.broadcast_to(cut, cut_ref.shape)


def _reduce_body(w_ref, b_ref, thr_ref, cut_ref, x_ref, k_ref,
                 feat_ref, o2_ref, acc_ref):
    bi = pl.program_id(0)
    t = pl.program_id(1)
    nt = pl.num_programs(1)

    @pl.when(t == 0)
    def _():
        acc_ref[...] = jnp.zeros_like(acc_ref)

    key = k_ref[0]
    wd = key.shape[1]
    thr = thr_ref[bi, 0]
    cut = cut_ref[bi, 0]
    iota = (jax.lax.broadcasted_iota(jnp.int32, key.shape, 0) * wd
            + jax.lax.broadcasted_iota(jnp.int32, key.shape, 1)
            + t * _TH * wd)
    sel = (key > thr) | ((key == thr) & (iota < cut))
    msk = sel.astype(jnp.float32)
    x = x_ref[0]
    acc_ref[...] += jnp.sum(x * msk[None], axis=1)

    @pl.when(t == nt - 1)
    def _():
        f = jnp.sum(acc_ref[...], axis=1) * (1.0 / _K)
        feat_ref[...] = f[None, None]
        o2_ref[bi, 0] = jnp.sum(f * w_ref[0]) + b_ref[0, 0]


def kernel(X, W, b):
    B, C, H, Wd = X.shape
    HW = H * Wd
    nth = H // _TH
    b2 = b.reshape(1, 1)

    keys = pl.pallas_call(
        _score_body,
        grid=(B, nth),
        in_specs=[
            pl.BlockSpec((1, C), lambda bi, ti: (0, 0)),
            pl.BlockSpec(memory_space=pltpu.SMEM),
            pl.BlockSpec((1, C, _TH, Wd), lambda bi, ti: (bi, 0, ti, 0)),
        ],
        out_specs=pl.BlockSpec((1, _TH, Wd), lambda bi, ti: (bi, ti, 0)),
        out_shape=jax.ShapeDtypeStruct((B, H, Wd), jnp.int32),
    )(W, b2, X)

    keys2 = keys.reshape(B, HW)

    thr, cut = pl.pallas_call(
        _select_body,
        grid=(1,),
        in_specs=[pl.BlockSpec((B, HW), lambda i: (0, 0))],
        out_specs=[
            pl.BlockSpec((B, 128), lambda i: (0, 0)),
            pl.BlockSpec((B, 128), lambda i: (0, 0)),
        ],
        out_shape=[
            jax.ShapeDtypeStruct((B, 128), jnp.int32),
            jax.ShapeDtypeStruct((B, 128), jnp.int32),
        ],
    )(keys2)

    feat, out2 = pl.pallas_call(
        _reduce_body,
        grid=(B, nth),
        in_specs=[
            pl.BlockSpec((1, C), lambda bi, ti: (0, 0)),
            pl.BlockSpec(memory_space=pltpu.SMEM),
            pl.BlockSpec(memory_space=pltpu.SMEM),
            pl.BlockSpec(memory_space=pltpu.SMEM),
            pl.BlockSpec((1, C, _TH, Wd), lambda bi, ti: (bi, 0, ti, 0)),
            pl.BlockSpec((1, _TH, Wd), lambda bi, ti: (bi, ti, 0)),
        ],
        out_specs=[
            pl.BlockSpec((1, 1, C), lambda bi, ti: (bi, 0, 0)),
            pl.BlockSpec((B, 1), lambda bi, ti: (0, 0),
                         memory_space=pltpu.SMEM),
        ],
        out_shape=[
            jax.ShapeDtypeStruct((B, 1, C), jnp.float32),
            jax.ShapeDtypeStruct((B, 1), jnp.float32),
        ],
        scratch_shapes=[pltpu.VMEM((C, Wd), jnp.float32)],
    )(W, b2, thr, cut, X, keys)

    return (feat.reshape(B, C), out2)
